# baseline (speedup 1.0000x reference)
import functools

import jax
import jax.numpy as jnp
import numpy as np
from jax import lax
from jax.experimental import pallas as pl
from jax.experimental.pallas import tpu as pltpu
from jax.experimental.pallas import tpu_sc as plsc

N, L, D = 256, 1025, 96
KEEP = 512
TOK = KEEP + 1
TOKP = 520
NB = 64
RING = 64
ADV = 16
O = -32
CC = 8
NCC = D // CC
LH = 128
NTASK = 2 * NCC


def _build_mask() -> np.ndarray:
    n = N * (L - 1)
    R = [[13, 15, 26, 6], [17, 29, 16, 24]]
    ks = [np.uint32(0), np.uint32(42),
          np.uint32(np.uint32(0) ^ np.uint32(42) ^ np.uint32(0x1BD11BDA))]
    with np.errstate(over="ignore"):
        x0 = np.full(n, ks[0], dtype=np.uint32)
        x1 = np.arange(n, dtype=np.uint32) + ks[1]
        for r in range(5):
            for d in R[r % 2]:
                x0 = x0 + x1
                x1 = (x1 << np.uint32(d)) | (x1 >> np.uint32(32 - d))
                x1 = x0 ^ x1
            x0 = x0 + ks[(r + 1) % 3]
            x1 = x1 + ks[(r + 2) % 3] + np.uint32(r + 1)
        bits = x0 ^ x1
    u = ((bits >> np.uint32(9)) | np.uint32(0x3F800000)).view(np.float32)
    u = np.maximum(u - np.float32(1.0), np.float32(0.0)).reshape(N, L - 1)
    pm = np.argsort(u, axis=1, kind="stable").astype(np.int32) + 1
    pm = np.sort(pm[:, :KEEP], axis=1)
    return np.concatenate([np.zeros((N, 1), np.int32), pm], axis=1)


_MASK = _build_mask()


def _ring_coverage():
    lo = np.empty(TOK, np.int64)
    hi = np.empty(TOK, np.int64)
    for j in range(TOK):
        b = min(j // 8, NB - 1)
        l, h = max(0, ADV * b + O), min(ADV * b + O + RING, L)
        if b <= 2:
            l, h = 0, RING
        if b == NB - 1:
            l, h = ADV * (NB - 1) + O + 1, L
        lo[j], hi[j] = l, h
    return lo, hi


_LO, _HI = _ring_coverage()
_BAD = (_MASK < _LO[None, :]) | (_MASK >= _HI[None, :])
_BAD_N, _BAD_J = np.nonzero(_BAD)
_BAD_T = _MASK[_BAD_N, _BAD_J]
_BAD_N = _BAD_N.astype(np.int32)
_BAD_J = _BAD_J.astype(np.int32)
_BAD_T = _BAD_T.astype(np.int32)

_TOKT = np.zeros((TOKP, N), np.int32)
_TOKT[:TOK] = _MASK.T

_MESH = plsc.VectorSubcoreMesh(core_axis_name="c", subcore_axis_name="s")
_PARAMS = pltpu.CompilerParams(
    use_tc_tiling_on_sc=True, needs_layout_passes=False
)
_OUT_T = jax.ShapeDtypeStruct((TOK, D, N), jnp.float32)


@functools.partial(
    pl.kernel,
    mesh=_MESH,
    compiler_params=_PARAMS,
    out_type=_OUT_T,
    scratch_types=[
        pltpu.VMEM((RING, CC, LH), jnp.float32),
        pltpu.VMEM((8, CC, LH), jnp.float32),
        pltpu.VMEM((8, LH), jnp.int32),
    ],
)
def _gather_t(xt_hbm, tokt_hbm, out_hbm, ring, outb, idxb):
    wid = lax.axis_index("s") * 2 + lax.axis_index("c")

    @pl.when(wid < NTASK)
    def task():
        half = wid % 2
        cc = wid // 2
        c0 = cc * CC
        l0 = half * LH
        lanes = [lax.iota(jnp.int32, 16) + 16 * g for g in range(LH // 16)]
        cvecs = [jnp.full((16,), c, jnp.int32) for c in range(CC)]

        pltpu.sync_copy(
            xt_hbm.at[pl.ds(0, RING), pl.ds(c0, CC), pl.ds(l0, LH)], ring
        )

        @pl.loop(0, NB)
        def per_batch(b):
            @pl.when(jnp.logical_and(b >= 3, b <= NB - 2))
            def slide():
                t0 = ADV * b + ADV
                s0 = pl.multiple_of(t0 % RING, ADV)
                pltpu.sync_copy(
                    xt_hbm.at[pl.ds(t0, ADV), pl.ds(c0, CC), pl.ds(l0, LH)],
                    ring.at[pl.ds(s0, ADV)],
                )

            @pl.when(b == NB - 1)
            def last_token():
                pltpu.sync_copy(
                    xt_hbm.at[pl.ds(L - 1, 1), pl.ds(c0, CC), pl.ds(l0, LH)],
                    ring.at[pl.ds(0, 1)],
                )

            pltpu.sync_copy(
                tokt_hbm.at[pl.ds(b * 8, 8), pl.ds(l0, LH)], idxb
            )
            for jr in range(8):
                for g in range(LH // 16):
                    slot = idxb[jr, pl.ds(16 * g, 16)] & (RING - 1)
                    for c in range(CC):
                        outb[jr, c, pl.ds(16 * g, 16)] = plsc.load_gather(
                            ring, [slot, cvecs[c], lanes[g]]
                        )
            pltpu.sync_copy(
                outb,
                out_hbm.at[pl.ds(b * 8, 8), pl.ds(c0, CC), pl.ds(l0, LH)],
            )

        pltpu.sync_copy(tokt_hbm.at[pl.ds(KEEP, 8), pl.ds(l0, LH)], idxb)
        for g in range(LH // 16):
            slot = idxb[0, pl.ds(16 * g, 16)] & (RING - 1)
            for c in range(CC):
                outb[0, c, pl.ds(16 * g, 16)] = plsc.load_gather(
                    ring, [slot, cvecs[c], lanes[g]]
                )
        pltpu.sync_copy(
            outb.at[pl.ds(0, 1)],
            out_hbm.at[pl.ds(KEEP, 1), pl.ds(c0, CC), pl.ds(l0, LH)],
        )


@functools.partial(
    pl.kernel,
    mesh=_MESH,
    compiler_params=_PARAMS,
    out_type=_OUT_T,
    scratch_types=[
        pltpu.VMEM((8, CC, LH), jnp.float32),
    ],
)
def _bcast_t(xt_hbm, tokt_hbm, out_hbm, outb):
    wid = lax.axis_index("s") * 2 + lax.axis_index("c")

    @pl.when(wid < NTASK)
    def task():
        half = wid % 2
        c0 = (wid // 2) * CC
        l0 = half * LH
        pltpu.sync_copy(
            xt_hbm.at[pl.ds(0, 1), pl.ds(c0, CC), pl.ds(l0, LH)],
            outb.at[pl.ds(0, 1)],
        )
        cls = [
            [outb[0, c, pl.ds(16 * g, 16)] for g in range(LH // 16)]
            for c in range(CC)
        ]
        for jr in range(1, 8):
            for c in range(CC):
                for g in range(LH // 16):
                    outb[jr, c, pl.ds(16 * g, 16)] = cls[c][g]

        @pl.loop(0, NB)
        def per_batch(b):
            pltpu.sync_copy(
                outb,
                out_hbm.at[pl.ds(b * 8, 8), pl.ds(c0, CC), pl.ds(l0, LH)],
            )

        pltpu.sync_copy(
            outb.at[pl.ds(0, 1)],
            out_hbm.at[pl.ds(KEEP, 1), pl.ds(c0, CC), pl.ds(l0, LH)],
        )


def kernel(x, force_drop):
    flag = (jnp.asarray(force_drop) != 0).astype(jnp.int32)
    xt = jnp.transpose(x, (1, 2, 0))
    tokt = jnp.asarray(_TOKT) * flag
    out_t = lax.cond(flag != 0, _gather_t, _bcast_t, xt, tokt)
    out = jnp.transpose(out_t, (2, 0, 1))
    bad_t = jnp.asarray(_BAD_T) * flag
    updates = x[jnp.asarray(_BAD_N), bad_t, :]
    return out.at[jnp.asarray(_BAD_N), jnp.asarray(_BAD_J), :].set(updates)


# speedup vs baseline: 3.0297x; 3.0297x over previous
import functools

import jax
import jax.numpy as jnp
import numpy as np
from jax import lax
from jax.experimental import pallas as pl
from jax.experimental.pallas import tpu as pltpu
from jax.experimental.pallas import tpu_sc as plsc

N, L, D = 256, 1025, 96
KEEP = 512
TOK = KEEP + 1
TOKP = 520
NW = 32
SPW = N // NW


def _build_mask() -> np.ndarray:
    n = N * (L - 1)
    R = [[13, 15, 26, 6], [17, 29, 16, 24]]
    ks = [np.uint32(0), np.uint32(42),
          np.uint32(np.uint32(0) ^ np.uint32(42) ^ np.uint32(0x1BD11BDA))]
    with np.errstate(over="ignore"):
        x0 = np.full(n, ks[0], dtype=np.uint32)
        x1 = np.arange(n, dtype=np.uint32) + ks[1]
        for r in range(5):
            for d in R[r % 2]:
                x0 = x0 + x1
                x1 = (x1 << np.uint32(d)) | (x1 >> np.uint32(32 - d))
                x1 = x0 ^ x1
            x0 = x0 + ks[(r + 1) % 3]
            x1 = x1 + ks[(r + 2) % 3] + np.uint32(r + 1)
        bits = x0 ^ x1
    u = ((bits >> np.uint32(9)) | np.uint32(0x3F800000)).view(np.float32)
    u = np.maximum(u - np.float32(1.0), np.float32(0.0)).reshape(N, L - 1)
    pm = np.argsort(u, axis=1, kind="stable").astype(np.int32) + 1
    pm = np.sort(pm[:, :KEEP], axis=1)
    return np.concatenate([np.zeros((N, 1), np.int32), pm], axis=1)


_MASK = _build_mask()

_TOKF = np.zeros((N, TOKP), np.int32)
_TOKF[:, :TOK] = _MASK
_TOKF = _TOKF.reshape(-1)

_MESH = plsc.VectorSubcoreMesh(core_axis_name="c", subcore_axis_name="s")
_PARAMS = pltpu.CompilerParams(use_tc_tiling_on_sc=False)


@functools.partial(
    pl.kernel,
    mesh=_MESH,
    compiler_params=_PARAMS,
    out_type=jax.ShapeDtypeStruct((N, TOK, D), jnp.float32),
    scratch_types=[
        pltpu.VMEM((2, TOKP), jnp.int32),
        pltpu.VMEM((2, TOKP, D), jnp.float32),
        pltpu.SemaphoreType.DMA,
        pltpu.SemaphoreType.DMA,
        pltpu.SemaphoreType.DMA,
        pltpu.SemaphoreType.DMA,
    ],
)
def _select(x_hbm, tok_hbm, out_hbm, idxv, rows, g0, g1, w0, w1):
    wid = lax.axis_index("s") * 2 + lax.axis_index("c")
    gsem = [g0, g1]
    wsem = [w0, w1]
    gather = [None] * SPW
    write = [None] * SPW
    for s in range(SPW):
        b = s % 2
        n = wid * SPW + s
        if s >= 2:
            write[s - 2].wait()
        pltpu.sync_copy(tok_hbm.at[pl.ds(n * TOKP, TOKP)], idxv.at[b])
        gather[s] = pltpu.async_copy(
            x_hbm.at[n].at[idxv.at[b]], rows.at[b], gsem[b]
        )
        if s >= 1:
            gather[s - 1].wait()
            write[s - 1] = pltpu.async_copy(
                rows.at[1 - b, pl.ds(0, TOK)], out_hbm.at[n - 1], wsem[1 - b]
            )
    last = SPW - 1
    gather[last].wait()
    write[last] = pltpu.async_copy(
        rows.at[last % 2, pl.ds(0, TOK)],
        out_hbm.at[wid * SPW + last],
        wsem[last % 2],
    )
    write[last - 1].wait()
    write[last].wait()


def kernel(x, force_drop):
    flag = (jnp.asarray(force_drop) != 0).astype(jnp.int32)
    tokf = jnp.asarray(_TOKF) * flag
    return _select(x, tokf)


# speedup vs baseline: 3.1283x; 1.0326x over previous
import functools

import jax
import jax.numpy as jnp
import numpy as np
from jax import lax
from jax.experimental import pallas as pl
from jax.experimental.pallas import tpu as pltpu
from jax.experimental.pallas import tpu_sc as plsc

N, L, D = 256, 1025, 96
KEEP = 512
TOK = KEEP + 1
TOKP = 520
NW = 32
SPW = N // NW


def _build_mask() -> np.ndarray:
    n = N * (L - 1)
    R = [[13, 15, 26, 6], [17, 29, 16, 24]]
    ks = [np.uint32(0), np.uint32(42),
          np.uint32(np.uint32(0) ^ np.uint32(42) ^ np.uint32(0x1BD11BDA))]
    with np.errstate(over="ignore"):
        x0 = np.full(n, ks[0], dtype=np.uint32)
        x1 = np.arange(n, dtype=np.uint32) + ks[1]
        for r in range(5):
            for d in R[r % 2]:
                x0 = x0 + x1
                x1 = (x1 << np.uint32(d)) | (x1 >> np.uint32(32 - d))
                x1 = x0 ^ x1
            x0 = x0 + ks[(r + 1) % 3]
            x1 = x1 + ks[(r + 2) % 3] + np.uint32(r + 1)
        bits = x0 ^ x1
    u = ((bits >> np.uint32(9)) | np.uint32(0x3F800000)).view(np.float32)
    u = np.maximum(u - np.float32(1.0), np.float32(0.0)).reshape(N, L - 1)
    pm = np.argsort(u, axis=1, kind="stable").astype(np.int32) + 1
    pm = np.sort(pm[:, :KEEP], axis=1)
    return np.concatenate([np.zeros((N, 1), np.int32), pm], axis=1)


_MASK = _build_mask()

_TOKF = np.zeros((N, TOKP), np.int64)
_TOKF[:, :TOK] = _MASK.astype(np.int64) * N
_TOKF = _TOKF.reshape(-1).astype(np.int32)
_NF = np.repeat(np.arange(N, dtype=np.int32), TOKP)

_MESH = plsc.VectorSubcoreMesh(core_axis_name="c", subcore_axis_name="s")
_PARAMS = pltpu.CompilerParams(use_tc_tiling_on_sc=False)


@functools.partial(
    pl.kernel,
    mesh=_MESH,
    compiler_params=_PARAMS,
    out_type=jax.ShapeDtypeStruct((N, TOK, D), jnp.float32),
    scratch_types=[
        pltpu.VMEM((2, TOKP), jnp.int32),
        pltpu.VMEM((2, TOKP, D), jnp.float32),
        pltpu.SemaphoreType.DMA,
        pltpu.SemaphoreType.DMA,
        pltpu.SemaphoreType.DMA,
        pltpu.SemaphoreType.DMA,
    ],
)
def _select(x_hbm, tok_hbm, out_hbm, idxv, rows, g0, g1, w0, w1):
    wid = lax.axis_index("s") * 2 + lax.axis_index("c")
    gsem = [g0, g1]
    wsem = [w0, w1]
    gather = [None] * SPW
    write = [None] * SPW
    for s in range(SPW):
        b = s % 2
        n = wid * SPW + s
        if s >= 2:
            write[s - 2].wait()
        pltpu.sync_copy(tok_hbm.at[pl.ds(n * TOKP, TOKP)], idxv.at[b])
        gather[s] = pltpu.async_copy(
            x_hbm.at[idxv.at[b]], rows.at[b], gsem[b]
        )
        if s >= 1:
            gather[s - 1].wait()
            write[s - 1] = pltpu.async_copy(
                rows.at[1 - b, pl.ds(0, TOK)], out_hbm.at[n - 1], wsem[1 - b]
            )
    last = SPW - 1
    gather[last].wait()
    write[last] = pltpu.async_copy(
        rows.at[last % 2, pl.ds(0, TOK)],
        out_hbm.at[wid * SPW + last],
        wsem[last % 2],
    )
    write[last - 1].wait()
    write[last].wait()


def kernel(x, force_drop):
    flag = (jnp.asarray(force_drop) != 0).astype(jnp.int32)
    gidx = jnp.asarray(_TOKF) * flag + jnp.asarray(_NF)
    table = jnp.transpose(x, (1, 0, 2)).reshape(L * N, D)
    return _select(table, gidx)
